# baseline (speedup 1.0000x reference)
import functools

import jax
import jax.numpy as jnp
from jax import lax
from jax.experimental import pallas as pl
from jax.experimental.pallas import tpu as pltpu
from jax.experimental.pallas import tpu_sc as plsc

_V = 1000
_NTOK = 1024 * 50
_NW = 32
_TPW = _NTOK // _NW
_CH = 32
_NCH = _TPW // _CH


def _sc_onehot(x_hbm, out_hbm, idx_v, buf0, buf1, sem0, sem1):
    wid = lax.axis_index("s") * 2 + lax.axis_index("c")
    base = wid * _TPW
    pltpu.sync_copy(x_hbm.at[pl.ds(base, _TPW)], idx_v)

    zeros = jnp.zeros((16,), jnp.float32)
    ones = jnp.ones((16,), jnp.float32)
    iota_sc = lax.iota(jnp.int32, 16) * _V

    def _zero(i, carry):
        buf0[pl.ds(i * 16, 16)] = zeros
        buf1[pl.ds(i * 16, 16)] = zeros
        return carry

    lax.fori_loop(0, _CH * _V // 16, _zero, 0)

    bufs = (buf0, buf1)
    sems = (sem0, sem1)
    handles = [None, None]
    for c in range(_NCH):
        b = c % 2
        buf = bufs[b]
        if handles[b] is not None:
            handles[b].wait()
            oc = c - 2
            for v in range(_CH // 16):
                ids = idx_v[pl.ds(oc * _CH + v * 16, 16)]
                pos = ids + iota_sc + (v * 16 * _V)
                plsc.store_scatter(buf, [pos], zeros)
        for v in range(_CH // 16):
            ids = idx_v[pl.ds(c * _CH + v * 16, 16)]
            pos = ids + iota_sc + (v * 16 * _V)
            plsc.store_scatter(buf, [pos], ones)
        handles[b] = pltpu.async_copy(
            buf, out_hbm.at[pl.ds((base + c * _CH) * _V, _CH * _V)], sems[b]
        )
    handles[0].wait()
    handles[1].wait()


def kernel(x):
    xi = x.reshape(-1).astype(jnp.int32)
    mesh = plsc.VectorSubcoreMesh(core_axis_name="c", subcore_axis_name="s")
    run = functools.partial(
        pl.kernel,
        mesh=mesh,
        out_type=jax.ShapeDtypeStruct((_NTOK * _V,), jnp.float32),
        scratch_types=[
            pltpu.VMEM((_TPW,), jnp.int32),
            pltpu.VMEM((_CH * _V,), jnp.float32),
            pltpu.VMEM((_CH * _V,), jnp.float32),
            pltpu.SemaphoreType.DMA,
            pltpu.SemaphoreType.DMA,
        ],
        compiler_params=pltpu.CompilerParams(needs_layout_passes=False),
    )(_sc_onehot)
    out = run(xi)
    return out.reshape(1024, 50, _V)
